# baseline (device time: 262809 ns/iter reference)
import jax
import jax.numpy as jnp
from jax import lax
from jax.experimental import pallas as pl
from jax.experimental.pallas import tpu as pltpu

N_DEV = 32


def kernel(x, w_mat, scale_x, scale_w):
    m_per, k = x.shape
    n_per = w_mat.shape[1]

    x16 = x.astype(jnp.bfloat16)
    w8 = w_mat.astype(jnp.float8_e4m3fn)

    def body(x_ref, w_ref, sx_ref, sw_ref, out_ref,
             w_all, res_buf, w_send, w_recv, res_send, res_recv):
        my = lax.axis_index("i")

        barrier_sem = pltpu.get_barrier_semaphore()
        for d in range(1, N_DEV):
            pl.semaphore_signal(barrier_sem, inc=1,
                                device_id=(lax.rem(my + d, N_DEV),),
                                device_id_type=pl.DeviceIdType.MESH)
        pl.semaphore_wait(barrier_sem, N_DEV - 1)

        scale = sx_ref[0] * sw_ref[0]
        xv = x_ref[...]

        desc_w = [None] * N_DEV
        for d in range(1, N_DEV):
            desc_w[d] = pltpu.make_async_remote_copy(
                src_ref=w_ref,
                dst_ref=w_all.at[my],
                send_sem=w_send.at[d],
                recv_sem=w_recv.at[d],
                device_id=(lax.rem(my + d, N_DEV),),
                device_id_type=pl.DeviceIdType.MESH,
            )
            desc_w[d].start()

        def mm(w16):
            acc = jnp.dot(xv, w16, preferred_element_type=jnp.float32)
            return jnp.maximum(acc * scale, 0.0)

        out_ref[pl.ds(my * m_per, m_per), :] = mm(w_ref[...].astype(jnp.bfloat16))

        desc_res = [None] * N_DEV
        for d in range(1, N_DEV):
            desc_w[d].wait_recv()
            o = lax.rem(my - d + N_DEV, N_DEV)
            res_buf[d] = mm(w_all[o].astype(jnp.bfloat16))
            desc_res[d] = pltpu.make_async_remote_copy(
                src_ref=res_buf.at[d],
                dst_ref=out_ref.at[pl.ds(my * m_per, m_per), :],
                send_sem=res_send.at[d],
                recv_sem=res_recv.at[d],
                device_id=(o,),
                device_id_type=pl.DeviceIdType.MESH,
            )
            desc_res[d].start()

        for d in range(1, N_DEV):
            desc_res[d].wait_recv()
        for d in range(1, N_DEV):
            desc_w[d].wait_send()
            desc_res[d].wait_send()

    return pl.pallas_call(
        body,
        out_shape=jax.ShapeDtypeStruct((N_DEV * m_per, n_per), jnp.float32),
        in_specs=[
            pl.BlockSpec(memory_space=pltpu.VMEM),
            pl.BlockSpec(memory_space=pltpu.VMEM),
            pl.BlockSpec(memory_space=pltpu.SMEM),
            pl.BlockSpec(memory_space=pltpu.SMEM),
        ],
        out_specs=pl.BlockSpec(memory_space=pltpu.VMEM),
        scratch_shapes=[
            pltpu.VMEM((N_DEV, k, n_per), jnp.float8_e4m3fn),
            pltpu.VMEM((N_DEV, m_per, n_per), jnp.float32),
            pltpu.SemaphoreType.DMA((N_DEV,)),
            pltpu.SemaphoreType.DMA((N_DEV,)),
            pltpu.SemaphoreType.DMA((N_DEV,)),
            pltpu.SemaphoreType.DMA((N_DEV,)),
        ],
        compiler_params=pltpu.CompilerParams(collective_id=0),
    )(x16, w8, scale_x, scale_w)


# device time: 215036 ns/iter; 1.2222x vs baseline; 1.2222x over previous
import jax
import jax.numpy as jnp
from jax import lax
from jax.experimental import pallas as pl
from jax.experimental.pallas import tpu as pltpu

N_DEV = 32
HR = N_DEV // 2
HL = N_DEV - 1 - HR


def kernel(x, w_mat, scale_x, scale_w):
    m_per, k = x.shape
    n_per = w_mat.shape[1]

    x16 = x.astype(jnp.bfloat16)
    w8 = w_mat.astype(jnp.float8_e4m3fn)

    def body(x_ref, w_ref, sx_ref, sw_ref, out_ref,
             buf_r, buf_l, res_r, res_l,
             send_r, recv_r, send_l, recv_l, res_send, res_recv):
        my = lax.axis_index("i")
        left = lax.rem(my - 1 + N_DEV, N_DEV)
        right = lax.rem(my + 1, N_DEV)

        barrier_sem = pltpu.get_barrier_semaphore()
        pl.semaphore_signal(barrier_sem, inc=1, device_id=(left,),
                            device_id_type=pl.DeviceIdType.MESH)
        pl.semaphore_signal(barrier_sem, inc=1, device_id=(right,),
                            device_id_type=pl.DeviceIdType.MESH)
        pl.semaphore_wait(barrier_sem, 2)

        scale = sx_ref[0] * sw_ref[0]
        xv = x_ref[...]

        def mm(w8_chunk):
            acc = jnp.dot(xv, w8_chunk.astype(jnp.bfloat16),
                          preferred_element_type=jnp.float32)
            return jnp.maximum(acc * scale, 0.0)

        my_rows = pl.ds(my * m_per, m_per)

        desc_r = [
            pltpu.make_async_remote_copy(
                src_ref=(w_ref if h == 1 else buf_r.at[h - 2]),
                dst_ref=buf_r.at[h - 1],
                send_sem=send_r.at[h - 1],
                recv_sem=recv_r.at[h - 1],
                device_id=(right,),
                device_id_type=pl.DeviceIdType.MESH,
            )
            for h in range(1, HR + 1)
        ]
        desc_l = [
            pltpu.make_async_remote_copy(
                src_ref=(w_ref if h == 1 else buf_l.at[h - 2]),
                dst_ref=buf_l.at[h - 1],
                send_sem=send_l.at[h - 1],
                recv_sem=recv_l.at[h - 1],
                device_id=(left,),
                device_id_type=pl.DeviceIdType.MESH,
            )
            for h in range(1, HL + 1)
        ]

        desc_r[0].start()
        desc_l[0].start()
        out_ref[my_rows, :] = mm(w_ref[...])

        desc_res = [None] * N_DEV
        for h in range(1, HR + 1):
            desc_r[h - 1].wait_recv()
            if h < HR:
                desc_r[h].start()
            if h <= HL:
                desc_l[h - 1].wait_recv()
                if h < HL:
                    desc_l[h].start()

            o = lax.rem(my - h + N_DEV, N_DEV)
            res_r[h - 1] = mm(buf_r[h - 1])
            desc_res[h] = pltpu.make_async_remote_copy(
                src_ref=res_r.at[h - 1],
                dst_ref=out_ref.at[my_rows, :],
                send_sem=res_send.at[h],
                recv_sem=res_recv.at[h],
                device_id=(o,),
                device_id_type=pl.DeviceIdType.MESH,
            )
            desc_res[h].start()

            if h <= HL:
                o2 = lax.rem(my + h, N_DEV)
                res_l[h - 1] = mm(buf_l[h - 1])
                desc_res[N_DEV - h] = pltpu.make_async_remote_copy(
                    src_ref=res_l.at[h - 1],
                    dst_ref=out_ref.at[my_rows, :],
                    send_sem=res_send.at[N_DEV - h],
                    recv_sem=res_recv.at[N_DEV - h],
                    device_id=(o2,),
                    device_id_type=pl.DeviceIdType.MESH,
                )
                desc_res[N_DEV - h].start()

        for lbl in range(1, N_DEV):
            desc_res[lbl].wait_recv()
        for d in desc_r + desc_l:
            d.wait_send()
        for lbl in range(1, N_DEV):
            desc_res[lbl].wait_send()

    return pl.pallas_call(
        body,
        out_shape=jax.ShapeDtypeStruct((N_DEV * m_per, n_per), jnp.float32),
        in_specs=[
            pl.BlockSpec(memory_space=pltpu.VMEM),
            pl.BlockSpec(memory_space=pltpu.VMEM),
            pl.BlockSpec(memory_space=pltpu.SMEM),
            pl.BlockSpec(memory_space=pltpu.SMEM),
        ],
        out_specs=pl.BlockSpec(memory_space=pltpu.VMEM),
        scratch_shapes=[
            pltpu.VMEM((HR, k, n_per), jnp.float8_e4m3fn),
            pltpu.VMEM((HL, k, n_per), jnp.float8_e4m3fn),
            pltpu.VMEM((HR, m_per, n_per), jnp.float32),
            pltpu.VMEM((HL, m_per, n_per), jnp.float32),
            pltpu.SemaphoreType.DMA((HR,)),
            pltpu.SemaphoreType.DMA((HR,)),
            pltpu.SemaphoreType.DMA((HL,)),
            pltpu.SemaphoreType.DMA((HL,)),
            pltpu.SemaphoreType.DMA((N_DEV,)),
            pltpu.SemaphoreType.DMA((N_DEV,)),
        ],
        compiler_params=pltpu.CompilerParams(collective_id=0),
    )(x16, w8, scale_x, scale_w)
